# baseline (speedup 1.0000x reference)
import functools

import jax
import jax.numpy as jnp
from jax import lax
from jax.experimental import pallas as pl
from jax.experimental.pallas import tpu as pltpu
from jax.experimental.pallas import tpu_sc as plsc

NC = 2
NS = 16
NW = NC * NS
CH = 128


@functools.cache
def _make_sc_gather(B, DU, DI, VU, VI):
    BPW = B // NW
    NCH = BPW // CH
    mesh = plsc.VectorSubcoreMesh(core_axis_name="c", subcore_axis_name="s")

    @functools.partial(
        pl.kernel,
        mesh=mesh,
        compiler_params=pltpu.CompilerParams(use_tc_tiling_on_sc=False),
        out_type=(
            jax.ShapeDtypeStruct((DU, B), jnp.float32),
            jax.ShapeDtypeStruct((DI, B), jnp.float32),
        ),
        scratch_types=[
            pltpu.VMEM((BPW,), jnp.int32),
            pltpu.VMEM((BPW,), jnp.int32),
            pltpu.VMEM((DU, BPW), jnp.float32),
            pltpu.VMEM((DI, BPW), jnp.float32),
            pltpu.SemaphoreType.DMA,
            pltpu.SemaphoreType.DMA,
        ],
    )
    def sc_gather(uid_hbm, iid_hbm, utT_hbm, itT_hbm, ueT_hbm, ieT_hbm,
                  uidx_v, iidx_v, ubuf_v, ibuf_v, sem_u, sem_i):
        wid = lax.axis_index("s") * NC + lax.axis_index("c")
        base = wid * BPW
        pltpu.sync_copy(uid_hbm.at[pl.ds(base, BPW)], uidx_v)
        pltpu.sync_copy(iid_hbm.at[pl.ds(base, BPW)], iidx_v)

        def gat(d, carry):
            for j in range(NCH):
                idx_u = uidx_v.at[pl.ds(j * CH, CH)]
                idx_i = iidx_v.at[pl.ds(j * CH, CH)]
                pltpu.async_copy(utT_hbm.at[d].at[idx_u],
                                 ubuf_v.at[d, pl.ds(j * CH, CH)], sem_u)
                pltpu.async_copy(itT_hbm.at[d].at[idx_i],
                                 ibuf_v.at[d, pl.ds(j * CH, CH)], sem_i)
            return carry

        lax.fori_loop(0, DU, gat, 0)
        pltpu.make_async_copy(ueT_hbm.at[:, pl.ds(0, BPW)], ubuf_v,
                              sem_u).wait()
        pltpu.make_async_copy(ieT_hbm.at[:, pl.ds(0, BPW)], ibuf_v,
                              sem_i).wait()
        pltpu.sync_copy(ubuf_v, ueT_hbm.at[:, pl.ds(base, BPW)])
        pltpu.sync_copy(ibuf_v, ieT_hbm.at[:, pl.ds(base, BPW)])

    return sc_gather


def _dense_body(ueT_ref, ieT_ref, uf_ref, wuf_ref, buf_ref, wt_ref, bt_ref,
                out_ref, *, DU):
    uft = jnp.dot(uf_ref[...], wuf_ref[...],
                  preferred_element_type=jnp.float32) + buf_ref[...]
    wt = wt_ref[...]
    cdim = (((0,), (0,)), ((), ()))
    acc = lax.dot_general(ueT_ref[...], wt[0:DU], cdim,
                          preferred_element_type=jnp.float32)
    acc = acc + jnp.dot(uft, wt[DU:2 * DU], preferred_element_type=jnp.float32)
    acc = acc + lax.dot_general(ieT_ref[...], wt[2 * DU:], cdim,
                                preferred_element_type=jnp.float32)
    out_ref[...] = acc + bt_ref[...]


@functools.cache
def _make_tc_dense(B, DU, DI, IU, T, BLK=2048):
    grid = B // BLK
    return pl.pallas_call(
        functools.partial(_dense_body, DU=DU),
        grid=(grid,),
        in_specs=[
            pl.BlockSpec((DU, BLK), lambda i: (0, i)),
            pl.BlockSpec((DI, BLK), lambda i: (0, i)),
            pl.BlockSpec((BLK, IU), lambda i: (i, 0)),
            pl.BlockSpec((IU, DU), lambda i: (0, 0)),
            pl.BlockSpec((1, DU), lambda i: (0, 0)),
            pl.BlockSpec((2 * DU + DI, T), lambda i: (0, 0)),
            pl.BlockSpec((1, T), lambda i: (0, 0)),
        ],
        out_specs=pl.BlockSpec((BLK, T), lambda i: (i, 0)),
        out_shape=jax.ShapeDtypeStruct((B, T), jnp.float32),
    )


def kernel(user_id, user_features, item_id, user_table, item_table,
           W_uf, b_uf, W_task, b_task):
    B = user_id.shape[0]
    VU, DU = user_table.shape
    VI, DI = item_table.shape
    IU = user_features.shape[1]
    T = W_task.shape[1]
    uid = user_id.astype(jnp.int32)
    iid = item_id.astype(jnp.int32)
    ueT, ieT = _make_sc_gather(B, DU, DI, VU, VI)(uid, iid, user_table.T,
                                                  item_table.T)
    return _make_tc_dense(B, DU, DI, IU, T)(
        ueT, ieT, user_features, W_uf,
        b_uf.reshape(1, DU), W_task, b_task.reshape(1, T))


# speedup vs baseline: 14.1873x; 14.1873x over previous
import functools

import jax
import jax.numpy as jnp
from jax import lax
from jax.experimental import pallas as pl
from jax.experimental.pallas import tpu as pltpu
from jax.experimental.pallas import tpu_sc as plsc

NC = 2
NS = 16
NW = NC * NS


XB = 4096


def _xpose_body(inT_ref, out_ref):
    x = inT_ref[...]
    h = XB // 2
    a = jnp.swapaxes(x[:, :h], 0, 1)
    b = jnp.swapaxes(x[:, h:], 0, 1)
    out_ref[...] = jnp.concatenate([a, b], axis=1)


@functools.cache
def _make_tc_xpose(V, D):
    grid = (V + XB - 1) // XB
    return pl.pallas_call(
        _xpose_body,
        grid=(grid,),
        in_specs=[pl.BlockSpec((D, XB), lambda i: (0, i))],
        out_specs=pl.BlockSpec((XB // 2, 2 * D), lambda i: (i, 0)),
        out_shape=jax.ShapeDtypeStruct((grid * (XB // 2), 2 * D),
                                       jnp.float32),
    )


@functools.cache
def _make_sc_gather(B, D2, VU, VI):
    BPW = B // NW
    HB = BPW // 4
    mesh = plsc.VectorSubcoreMesh(core_axis_name="c", subcore_axis_name="s")

    @functools.partial(
        pl.kernel,
        mesh=mesh,
        out_type=(
            jax.ShapeDtypeStruct((B, D2), jnp.float32),
            jax.ShapeDtypeStruct((B, D2), jnp.float32),
        ),
        scratch_types=[
            pltpu.VMEM((BPW,), jnp.int32),
            pltpu.VMEM((BPW,), jnp.int32),
            pltpu.VMEM((HB, D2), jnp.float32),
            pltpu.VMEM((HB, D2), jnp.float32),
            pltpu.SemaphoreType.DMA,
            pltpu.SemaphoreType.DMA,
        ],
    )
    def sc_gather(uid_hbm, iid_hbm, utab_hbm, itab_hbm, ue_hbm, ie_hbm,
                  uidx_v, iidx_v, urows_v, irows_v, sem_u, sem_i):
        wid = lax.axis_index("s") * NC + lax.axis_index("c")
        base = wid * BPW
        pltpu.sync_copy(uid_hbm.at[pl.ds(base, BPW)], uidx_v)
        pltpu.sync_copy(iid_hbm.at[pl.ds(base, BPW)], iidx_v)

        for h in range(4):
            def body_u(c, carry):
                s0 = h * HB + c * 16
                d0 = c * 16
                uvec = uidx_v[pl.ds(s0, 16)]
                for j in range(16):
                    pltpu.async_copy(utab_hbm.at[pl.ds(uvec[j], 1)],
                                     urows_v.at[pl.ds(d0 + j, 1)], sem_u)
                return carry

            def body_i(c, carry):
                s0 = h * HB + c * 16
                d0 = c * 16
                ivec = iidx_v[pl.ds(s0, 16)]
                for j in range(16):
                    pltpu.async_copy(itab_hbm.at[pl.ds(ivec[j], 1)],
                                     irows_v.at[pl.ds(d0 + j, 1)], sem_i)
                return carry

            lax.fori_loop(0, HB // 16, body_u, 0)
            lax.fori_loop(0, HB // 16, body_i, 0)
            pltpu.make_async_copy(utab_hbm.at[pl.ds(0, HB)], urows_v,
                                  sem_u).wait()
            pltpu.make_async_copy(itab_hbm.at[pl.ds(0, HB)], irows_v,
                                  sem_i).wait()

            WB = 64

            def wb(k, carry):
                r0 = pl.multiple_of(k * WB, WB)
                pltpu.sync_copy(urows_v.at[pl.ds(r0, WB)],
                                ue_hbm.at[pl.ds(base + h * HB + r0, WB)])
                pltpu.sync_copy(irows_v.at[pl.ds(r0, WB)],
                                ie_hbm.at[pl.ds(base + h * HB + r0, WB)])
                return carry

            lax.fori_loop(0, HB // WB, wb, 0)

    return sc_gather


def _dense_body(ue2_ref, ie2_ref, selu_ref, seli_ref, uf_ref, wuf_ref,
                buf_ref, wt_ref, bt_ref, out_ref, *, DU):
    ue2 = ue2_ref[...]
    ie2 = ie2_ref[...]
    ue = jnp.where(selu_ref[...] > 0, ue2[:, DU:], ue2[:, :DU])
    ie = jnp.where(seli_ref[...] > 0, ie2[:, DU:], ie2[:, :DU])
    uft = jnp.dot(uf_ref[...], wuf_ref[...],
                  preferred_element_type=jnp.float32) + buf_ref[...]
    wt = wt_ref[...]
    acc = jnp.dot(ue, wt[0:DU], preferred_element_type=jnp.float32)
    acc = acc + jnp.dot(uft, wt[DU:2 * DU], preferred_element_type=jnp.float32)
    acc = acc + jnp.dot(ie, wt[2 * DU:], preferred_element_type=jnp.float32)
    out_ref[...] = acc + bt_ref[...]


@functools.cache
def _make_tc_dense(B, DU, DI, IU, T, BLK=2048):
    grid = B // BLK
    return pl.pallas_call(
        functools.partial(_dense_body, DU=DU),
        grid=(grid,),
        in_specs=[
            pl.BlockSpec((BLK, 2 * DU), lambda i: (i, 0)),
            pl.BlockSpec((BLK, 2 * DI), lambda i: (i, 0)),
            pl.BlockSpec((BLK, 1), lambda i: (i, 0)),
            pl.BlockSpec((BLK, 1), lambda i: (i, 0)),
            pl.BlockSpec((BLK, IU), lambda i: (i, 0)),
            pl.BlockSpec((IU, DU), lambda i: (0, 0)),
            pl.BlockSpec((1, DU), lambda i: (0, 0)),
            pl.BlockSpec((2 * DU + DI, T), lambda i: (0, 0)),
            pl.BlockSpec((1, T), lambda i: (0, 0)),
        ],
        out_specs=pl.BlockSpec((BLK, T), lambda i: (i, 0)),
        out_shape=jax.ShapeDtypeStruct((B, T), jnp.float32),
    )


def kernel(user_id, user_features, item_id, user_table, item_table,
           W_uf, b_uf, W_task, b_task):
    B = user_id.shape[0]
    VU, DU = user_table.shape
    VI, DI = item_table.shape
    IU = user_features.shape[1]
    T = W_task.shape[1]
    uid = user_id.astype(jnp.int32)
    iid = item_id.astype(jnp.int32)
    upk = _make_tc_xpose(VU, DU)(user_table.T)
    ipk = _make_tc_xpose(VI, DI)(item_table.T)
    h = XB // 2
    gid_u = ((uid // XB) * h) + (uid & (h - 1))
    gid_i = ((iid // XB) * h) + (iid & (h - 1))
    ue2, ie2 = _make_sc_gather(B, 2 * DU, VU, VI)(gid_u, gid_i, upk, ipk)
    selu = ((uid // h) & 1).astype(jnp.float32).reshape(B, 1)
    seli = ((iid // h) & 1).astype(jnp.float32).reshape(B, 1)
    return _make_tc_dense(B, DU, DI, IU, T)(
        ue2, ie2, selu, seli, user_features, W_uf,
        b_uf.reshape(1, DU), W_task, b_task.reshape(1, T))
